# baseline (speedup 1.0000x reference)
import functools

import jax
import jax.numpy as jnp
from jax import lax
from jax.experimental import pallas as pl
from jax.experimental.pallas import tpu as pltpu
from jax.experimental.pallas import tpu_sc as plsc

SEQ = 8192
DIM = 2048
TOT = 4 * 8192
NC, NS = 2, 16
NW = NC * NS
PER_W = TOT // NW
CHUNK = 8
NBUF = 2
A_ROWS = PER_W // 2
B_ROWS = PER_W - A_ROWS
NA = A_ROWS // CHUNK
NB = B_ROWS // CHUNK
NG = NA // NBUF

_mesh = plsc.VectorSubcoreMesh(core_axis_name="c", subcore_axis_name="s")


@functools.partial(
    pl.kernel,
    out_type=jax.ShapeDtypeStruct((TOT, DIM), jnp.float32),
    mesh=_mesh,
    scratch_types=[
        pltpu.VMEM((PER_W,), jnp.int32),
        [pltpu.VMEM((CHUNK, DIM), jnp.float32)] * NBUF,
        pltpu.VMEM_SHARED((NS * NBUF * CHUNK, DIM), jnp.float32),
        [pltpu.SemaphoreType.DMA] * NBUF,
        [pltpu.SemaphoreType.DMA] * NBUF,
        [pltpu.SemaphoreType.DMA] * NBUF,
        [pltpu.SemaphoreType.DMA] * NBUF,
    ],
)
def _gather_sc(ids_hbm, table_hbm, out_hbm, idx_v, abufs, stage,
               gsa, psa, gsb, psb):
    wid = lax.axis_index("s") * NC + lax.axis_index("c")
    sid = lax.axis_index("s")
    base = wid * PER_W
    base_b = base + A_ROWS

    pltpu.sync_copy(ids_hbm.at[wid], idx_v)

    def idx_slice(j):
        return idx_v.at[pl.ds(j * CHUNK, CHUNK)]

    def gather_a(j, b):
        pltpu.async_copy(table_hbm.at[idx_slice(j)], abufs[b], gsa[b])

    def gwait_a(b):
        pltpu.make_async_copy(table_hbm.at[idx_slice(0)], abufs[b], gsa[b]).wait()

    def put_a(j, b):
        dst = out_hbm.at[pl.ds(base + j * CHUNK, CHUNK)]
        pltpu.async_copy(abufs[b], dst, psa[b])

    def pwait_a(b):
        dst = out_hbm.at[pl.ds(base, CHUNK)]
        pltpu.make_async_copy(abufs[b], dst, psa[b]).wait()

    def slab(b):
        return stage.at[pl.ds((sid * NBUF + b) * CHUNK, CHUNK)]

    def gather_b(j, b):
        vec = idx_v[pl.ds((NA + j) * CHUNK, CHUNK)]
        sl = slab(b)
        for k in range(CHUNK):
            pltpu.async_copy(
                table_hbm.at[pl.ds(vec[k], 1)], sl.at[pl.ds(k, 1)], gsb[b]
            )

    def gwait_b(b):
        sl = slab(b)
        for k in range(CHUNK):
            pltpu.make_async_copy(
                table_hbm.at[pl.ds(0, 1)], sl.at[pl.ds(k, 1)], gsb[b]
            ).wait()

    def put_b(j, b):
        dst = out_hbm.at[pl.ds(base_b + j * CHUNK, CHUNK)]
        pltpu.async_copy(slab(b), dst, psb[b])

    def pwait_b(b):
        dst = out_hbm.at[pl.ds(base_b, CHUNK)]
        pltpu.make_async_copy(slab(b), dst, psb[b]).wait()

    for b in range(NBUF):
        gather_a(b, b)
        gather_b(b, b)

    def body(g, carry):
        j0 = g * NBUF
        for b in range(NBUF):
            gwait_a(b)
            put_a(j0 + b, b)
            gwait_b(b)
            put_b(j0 + b, b)
        for b in range(NBUF):
            pwait_a(b)
            gather_a(j0 + NBUF + b, b)
            pwait_b(b)
            gather_b(j0 + NBUF + b, b)
        return carry

    lax.fori_loop(0, NG - 1, body, 0)

    j0 = (NG - 1) * NBUF
    for b in range(NBUF):
        gwait_a(b)
        put_a(j0 + b, b)
        gwait_b(b)
        put_b(j0 + b, b)
    for b in range(NBUF):
        pwait_a(b)
        pwait_b(b)


def kernel(position_ids, table):
    ids = position_ids.reshape(NW, PER_W).astype(jnp.int32)
    out = _gather_sc(ids, table)
    return out.reshape(position_ids.shape[0], position_ids.shape[1], DIM)
